# baseline (speedup 1.0000x reference)
import functools
import math

import jax
import jax.numpy as jnp
from jax import lax
from jax.experimental import pallas as pl
from jax.experimental.pallas import tpu as pltpu
from jax.experimental.pallas import tpu_sc as plsc

VOCAB = 1_000_000
VPACK = 500_032
D = 64
ROWS = 4096
COLS = 200
NC, NS = 2, 16
NW = NC * NS
BCOL = ROWS // NW
SCALE = math.sqrt(D)

_mesh = plsc.VectorSubcoreMesh(core_axis_name="c", subcore_axis_name="s")


def _pack_body(in_ref, out_ref):
    for i in range(8):
        t = in_ref[:, i * 128:(i + 1) * 128].T
        out_ref[i * 64:(i + 1) * 64, :] = jnp.concatenate(
            [t[0:64, :], t[64:128, :]], axis=1)


def _pack_table(tab_t):
    return pl.pallas_call(
        _pack_body,
        grid=(977,),
        in_specs=[pl.BlockSpec((64, 1024), lambda c: (0, c))],
        out_specs=pl.BlockSpec((512, 128), lambda c: (c, 0)),
        out_shape=jax.ShapeDtypeStruct((VPACK, 2 * D), jnp.float32),
    )(tab_t)


@functools.partial(
    pl.kernel,
    mesh=_mesh,
    compiler_params=pltpu.CompilerParams(use_tc_tiling_on_sc=False),
    out_type=jax.ShapeDtypeStruct((ROWS * COLS, 2 * D), jnp.float32),
    scratch_types=[
        pltpu.VMEM((COLS, BCOL), jnp.int32),
        pltpu.VMEM((BCOL, D), jnp.float32),
        pltpu.SemaphoreType.DMA,
    ],
)
def _gather(xs_hbm, tab_hbm, out_hbm, idx_v, rows_v, sem):
    wid = lax.axis_index("s") * NC + lax.axis_index("c")
    pltpu.sync_copy(xs_hbm.at[:, pl.ds(wid * BCOL, BCOL)], idx_v)

    def chunk_body(s, carry):
        pltpu.async_copy(tab_hbm.at[idx_v.at[s]], rows_v, sem).wait()
        base = s * ROWS + wid * BCOL
        pltpu.sync_copy(rows_v, out_hbm.at[pl.ds(base, BCOL), pl.ds(0, D)])
        return carry

    lax.fori_loop(0, COLS, chunk_body, 0)


def _finish_body(in_ref, out_ref):
    for b in range(NW):
        x = in_ref[b * BCOL:(b + 1) * BCOL, 0:D]
        out_ref[0, :, b * BCOL:(b + 1) * BCOL] = x.T * jnp.float32(SCALE)


def _finish(gathered):
    return pl.pallas_call(
        _finish_body,
        grid=(COLS,),
        in_specs=[pl.BlockSpec((ROWS, 2 * D), lambda s: (s, 0))],
        out_specs=pl.BlockSpec((1, D, ROWS), lambda s: (s, 0, 0)),
        out_shape=jax.ShapeDtypeStruct((COLS, D, ROWS), jnp.float32),
    )(gathered)


def kernel(x, table):
    xt = x.T.astype(jnp.int32)
    xs = ((xt & ~jnp.int32(127)) | ((xt & 63) << 1) | ((xt >> 6) & 1))

    tab_t = table.T
    packed = _pack_table(tab_t)
    tab_lin = packed.reshape(-1).reshape(2 * VPACK, D)

    gathered = _gather(xs, tab_lin)
    y = _finish(gathered)
    return jnp.transpose(y, (2, 0, 1))


# speedup vs baseline: 1.0905x; 1.0905x over previous
import functools
import math

import jax
import jax.numpy as jnp
from jax import lax
from jax.experimental import pallas as pl
from jax.experimental.pallas import tpu as pltpu
from jax.experimental.pallas import tpu_sc as plsc

VOCAB = 1_000_000
VPACK = 500_032
D = 64
ROWS = 4096
COLS = 200
NC, NS = 2, 16
NW = NC * NS
BCOL = ROWS // NW
SCALE = math.sqrt(D)

_mesh = plsc.VectorSubcoreMesh(core_axis_name="c", subcore_axis_name="s")


def _pack_body(in_ref, out_ref):
    for i in range(8):
        t = in_ref[:, i * 128:(i + 1) * 128].T
        out_ref[i * 64:(i + 1) * 64, :] = jnp.concatenate(
            [t[0:64, :], t[64:128, :]], axis=1)


def _pack_table(tab_t):
    return pl.pallas_call(
        _pack_body,
        grid=(977,),
        in_specs=[pl.BlockSpec((64, 1024), lambda c: (0, c))],
        out_specs=pl.BlockSpec((512, 128), lambda c: (c, 0)),
        out_shape=jax.ShapeDtypeStruct((VPACK, 2 * D), jnp.float32),
    )(tab_t)


@functools.partial(
    pl.kernel,
    mesh=_mesh,
    compiler_params=pltpu.CompilerParams(use_tc_tiling_on_sc=False),
    out_type=jax.ShapeDtypeStruct((ROWS * COLS, D), jnp.float32),
    scratch_types=[
        pltpu.VMEM((COLS, BCOL), jnp.int32),
        pltpu.VMEM((BCOL, D), jnp.float32),
        pltpu.VMEM((BCOL, D), jnp.float32),
        pltpu.SemaphoreType.DMA,
        pltpu.SemaphoreType.DMA,
    ],
)
def _gather(xs_hbm, tab_hbm, out_hbm, idx_v, rows0_v, rows1_v, sem0, sem1):
    wid = lax.axis_index("s") * NC + lax.axis_index("c")
    pltpu.sync_copy(xs_hbm.at[:, pl.ds(wid * BCOL, BCOL)], idx_v)

    bufs = (rows0_v, rows1_v)
    sems = (sem0, sem1)

    def gat(s, b):
        return pltpu.make_async_copy(tab_hbm.at[idx_v.at[s]], bufs[b], sems[b])

    def put(s, b):
        base = s * ROWS + wid * BCOL
        pltpu.sync_copy(bufs[b], out_hbm.at[pl.ds(base, BCOL)])

    gat(0, 0).start()
    gat(1, 1).start()

    def chunk_body(s2, carry):
        for b in range(2):
            s = s2 * 2 + b
            gat(s, b).wait()
            put(s, b)
            gat(s + 2, b).start()
        return carry

    lax.fori_loop(0, COLS // 2 - 1, chunk_body, 0)
    for b in range(2):
        s = COLS - 2 + b
        gat(s, b).wait()
        put(s, b)


def _finish_body(in_ref, out_ref):
    for k in range(NW):
        x = in_ref[k * D:(k + 1) * D, :]
        t = x.T
        out_ref[0, :, k * BCOL:(k + 1) * BCOL] = jnp.concatenate(
            [t[0:D, :], t[D:2 * D, :]], axis=1) * jnp.float32(SCALE)


def _finish(gathered):
    return pl.pallas_call(
        _finish_body,
        grid=(COLS,),
        in_specs=[pl.BlockSpec((ROWS * D // 128, 128), lambda s: (s, 0))],
        out_specs=pl.BlockSpec((1, D, ROWS), lambda s: (s, 0, 0)),
        out_shape=jax.ShapeDtypeStruct((COLS, D, ROWS), jnp.float32),
    )(gathered)


def kernel(x, table):
    xt = x.T.astype(jnp.int32)
    xs = ((xt & ~jnp.int32(127)) | ((xt & 63) << 1) | ((xt >> 6) & 1))
    xs = xs.reshape(COLS, NW, 2, D).transpose(0, 1, 3, 2).reshape(COLS, ROWS)

    tab_t = table.T
    packed = _pack_table(tab_t)
    tab_lin = packed.reshape(-1).reshape(2 * VPACK, D)

    gathered = _gather(xs, tab_lin)
    g2 = gathered.reshape(-1).reshape(ROWS * COLS * D // 128, 128)
    y = _finish(g2)
    return jnp.transpose(y, (2, 0, 1))


# speedup vs baseline: 1.8870x; 1.7303x over previous
import functools
import math

import jax
import jax.numpy as jnp
from jax import lax
from jax.experimental import pallas as pl
from jax.experimental.pallas import tpu as pltpu
from jax.experimental.pallas import tpu_sc as plsc

VOCAB = 1_000_000
VPACK = 500_032
D = 64
ROWS = 4096
COLS = 200
NC, NS = 2, 16
NW = NC * NS
BCOL = ROWS // NW
SCALE = math.sqrt(D)

_mesh = plsc.VectorSubcoreMesh(core_axis_name="c", subcore_axis_name="s")


def _pack_body(in_ref, out_ref):
    for i in range(64):
        t = in_ref[:, i * 128:(i + 1) * 128].T
        out_ref[i * 64:(i + 1) * 64, :] = jnp.concatenate(
            [t[0:64, :], t[64:128, :]], axis=1)


def _pack_table(tab_t):
    return pl.pallas_call(
        _pack_body,
        grid=(123,),
        in_specs=[pl.BlockSpec((64, 8192), lambda c: (0, c))],
        out_specs=pl.BlockSpec((4096, 128), lambda c: (c, 0)),
        out_shape=jax.ShapeDtypeStruct((VPACK, 2 * D), jnp.float32),
    )(tab_t)


@functools.partial(
    pl.kernel,
    mesh=_mesh,
    compiler_params=pltpu.CompilerParams(use_tc_tiling_on_sc=False),
    out_type=jax.ShapeDtypeStruct((ROWS * COLS, D), jnp.float32),
    scratch_types=[
        pltpu.VMEM((COLS, BCOL), jnp.int32),
        pltpu.VMEM((BCOL, D), jnp.float32),
        pltpu.VMEM((BCOL, D), jnp.float32),
        pltpu.SemaphoreType.DMA,
        pltpu.SemaphoreType.DMA,
    ],
)
def _gather(xs_hbm, tab_hbm, out_hbm, idx_v, rows0_v, rows1_v, sem0, sem1):
    wid = lax.axis_index("s") * NC + lax.axis_index("c")
    pltpu.sync_copy(xs_hbm.at[:, pl.ds(wid * BCOL, BCOL)], idx_v)

    bufs = (rows0_v, rows1_v)
    sems = (sem0, sem1)

    def gat(s, b):
        return pltpu.make_async_copy(tab_hbm.at[idx_v.at[s]], bufs[b], sems[b])

    def put(s, b):
        base = s * ROWS + wid * BCOL
        pltpu.sync_copy(bufs[b], out_hbm.at[pl.ds(base, BCOL)])

    gat(0, 0).start()
    gat(1, 1).start()

    def chunk_body(s2, carry):
        for b in range(2):
            s = s2 * 2 + b
            gat(s, b).wait()
            put(s, b)
            gat(s + 2, b).start()
        return carry

    lax.fori_loop(0, COLS // 2 - 1, chunk_body, 0)
    for b in range(2):
        s = COLS - 2 + b
        gat(s, b).wait()
        put(s, b)


def _finish_body(in_ref, out_ref):
    for s in range(2):
        for k in range(NW):
            r = s * NW + k
            x = in_ref[r * D:(r + 1) * D, :]
            t = x.T
            out_ref[s, :, k * BCOL:(k + 1) * BCOL] = jnp.concatenate(
                [t[0:D, :], t[D:2 * D, :]], axis=1) * jnp.float32(SCALE)


def _finish(gathered):
    return pl.pallas_call(
        _finish_body,
        grid=(COLS // 2,),
        in_specs=[pl.BlockSpec((ROWS * D * 2 // 128, 128), lambda s: (s, 0))],
        out_specs=pl.BlockSpec((2, D, ROWS), lambda s: (s, 0, 0)),
        out_shape=jax.ShapeDtypeStruct((COLS, D, ROWS), jnp.float32),
    )(gathered)


def kernel(x, table):
    xt = x.T.astype(jnp.int32)
    xs = ((xt & ~jnp.int32(127)) | ((xt & 63) << 1) | ((xt >> 6) & 1))
    xs = xs.reshape(COLS, NW, 2, D).transpose(0, 1, 3, 2).reshape(COLS, ROWS)

    tab_t = table.T
    packed = _pack_table(tab_t)
    tab_lin = packed.reshape(-1).reshape(2 * VPACK, D)

    gathered = _gather(xs, tab_lin)
    g2 = gathered.reshape(-1).reshape(ROWS * COLS * D // 128, 128)
    y = _finish(g2)
    return jnp.transpose(y, (2, 0, 1))


# speedup vs baseline: 2.1679x; 1.1489x over previous
import functools
import math

import jax
import jax.numpy as jnp
from jax import lax
from jax.experimental import pallas as pl
from jax.experimental.pallas import tpu as pltpu
from jax.experimental.pallas import tpu_sc as plsc

VOCAB = 1_000_000
VPACK = 500_032
D = 64
ROWS = 4096
COLS = 200
NC, NS = 2, 16
NW = NC * NS
BCOL = ROWS // NW
SCALE = math.sqrt(D)

_mesh = plsc.VectorSubcoreMesh(core_axis_name="c", subcore_axis_name="s")


def _pack_body(in_ref, out_ref):
    for i in range(128):
        t = in_ref[:, i * 128:(i + 1) * 128].T
        out_ref[i * 64:(i + 1) * 64, :] = jnp.concatenate(
            [t[0:64, :], t[64:128, :]], axis=1)


def _pack_table(tab_t):
    return pl.pallas_call(
        _pack_body,
        grid=(62,),
        in_specs=[pl.BlockSpec((64, 16384), lambda c: (0, c))],
        out_specs=pl.BlockSpec((8192, 128), lambda c: (c, 0)),
        out_shape=jax.ShapeDtypeStruct((VPACK, 2 * D), jnp.float32),
    )(tab_t)


@functools.partial(
    pl.kernel,
    mesh=_mesh,
    compiler_params=pltpu.CompilerParams(use_tc_tiling_on_sc=False),
    out_type=jax.ShapeDtypeStruct((ROWS * COLS, D), jnp.float32),
    scratch_types=[
        pltpu.VMEM((COLS, BCOL), jnp.int32),
        pltpu.VMEM((BCOL, D), jnp.float32),
        pltpu.VMEM((BCOL, D), jnp.float32),
        pltpu.VMEM((BCOL, D), jnp.float32),
        pltpu.SemaphoreType.DMA,
        pltpu.SemaphoreType.DMA,
        pltpu.SemaphoreType.DMA,
    ],
)
def _gather(xs_hbm, tab_hbm, out_hbm, idx_v, rows0_v, rows1_v, rows2_v,
            sem0, sem1, sem2):
    wid = lax.axis_index("s") * NC + lax.axis_index("c")
    pltpu.sync_copy(xs_hbm.at[:, pl.ds(wid * BCOL, BCOL)], idx_v)

    bufs = (rows0_v, rows1_v, rows2_v)
    sems = (sem0, sem1, sem2)

    def gat(s, b):
        return pltpu.make_async_copy(tab_hbm.at[idx_v.at[s]], bufs[b], sems[b])

    def put(s, b):
        base = s * ROWS + wid * BCOL
        pltpu.sync_copy(bufs[b], out_hbm.at[pl.ds(base, BCOL)])

    for b in range(3):
        gat(b, b).start()

    def chunk_body(s3, carry):
        for b in range(3):
            s = s3 * 3 + b
            gat(s, b).wait()
            put(s, b)
            gat(s + 3, b).start()
        return carry

    lax.fori_loop(0, 65, chunk_body, 0)
    for s in range(195, 197):
        b = s % 3
        gat(s, b).wait()
        put(s, b)
        gat(s + 3, b).start()
    for s in range(197, 200):
        b = s % 3
        gat(s, b).wait()
        put(s, b)


def _finish_body(in_ref, out_ref):
    for s in range(4):
        for k in range(NW):
            r = s * NW + k
            x = in_ref[r * D:(r + 1) * D, :]
            t = x.T
            out_ref[s, :, k * BCOL:(k + 1) * BCOL] = jnp.concatenate(
                [t[0:D, :], t[D:2 * D, :]], axis=1) * jnp.float32(SCALE)


def _finish(gathered):
    return pl.pallas_call(
        _finish_body,
        grid=(COLS // 4,),
        in_specs=[pl.BlockSpec((ROWS * D * 4 // 128, 128), lambda s: (s, 0))],
        out_specs=pl.BlockSpec((4, D, ROWS), lambda s: (s, 0, 0)),
        out_shape=jax.ShapeDtypeStruct((COLS, D, ROWS), jnp.float32),
    )(gathered)


def kernel(x, table):
    xt = x.T.astype(jnp.int32)
    xs = ((xt & ~jnp.int32(127)) | ((xt & 63) << 1) | ((xt >> 6) & 1))
    xs = xs.reshape(COLS, NW, 2, D).transpose(0, 1, 3, 2).reshape(COLS, ROWS)

    tab_t = table.T
    packed = _pack_table(tab_t)
    tab_lin = packed.reshape(-1).reshape(2 * VPACK, D)

    gathered = _gather(xs, tab_lin)
    g2 = gathered.reshape(-1).reshape(ROWS * COLS * D // 128, 128)
    y = _finish(g2)
    return jnp.transpose(y, (2, 0, 1))


# speedup vs baseline: 2.2662x; 1.0453x over previous
import functools
import math

import jax
import jax.numpy as jnp
from jax import lax
from jax.experimental import pallas as pl
from jax.experimental.pallas import tpu as pltpu
from jax.experimental.pallas import tpu_sc as plsc

VOCAB = 1_000_000
VPACK = 500_032
D = 64
ROWS = 4096
COLS = 200
NC, NS = 2, 16
NW = NC * NS
BCOL = ROWS // NW
SCALE = math.sqrt(D)

_mesh = plsc.VectorSubcoreMesh(core_axis_name="c", subcore_axis_name="s")


def _pack_body(in_ref, out_ref):
    for i in range(256):
        t = in_ref[:, i * 128:(i + 1) * 128].T
        out_ref[i * 64:(i + 1) * 64, :] = jnp.concatenate(
            [t[0:64, :], t[64:128, :]], axis=1)


def _pack_table(tab_t):
    return pl.pallas_call(
        _pack_body,
        grid=(31,),
        in_specs=[pl.BlockSpec((64, 32768), lambda c: (0, c))],
        out_specs=pl.BlockSpec((16384, 128), lambda c: (c, 0)),
        out_shape=jax.ShapeDtypeStruct((VPACK, 2 * D), jnp.float32),
    )(tab_t)


@functools.partial(
    pl.kernel,
    mesh=_mesh,
    compiler_params=pltpu.CompilerParams(use_tc_tiling_on_sc=False),
    out_type=jax.ShapeDtypeStruct((ROWS * COLS, D), jnp.float32),
    scratch_types=[
        pltpu.VMEM((COLS, BCOL), jnp.int32),
        pltpu.VMEM((BCOL, D), jnp.float32),
        pltpu.VMEM((BCOL, D), jnp.float32),
        pltpu.VMEM((BCOL, D), jnp.float32),
        pltpu.SemaphoreType.DMA,
        pltpu.SemaphoreType.DMA,
        pltpu.SemaphoreType.DMA,
    ],
)
def _gather(xs_hbm, tab_hbm, out_hbm, idx_v, rows0_v, rows1_v, rows2_v,
            sem0, sem1, sem2):
    wid = lax.axis_index("s") * NC + lax.axis_index("c")
    pltpu.sync_copy(xs_hbm.at[:, pl.ds(wid * BCOL, BCOL)], idx_v)

    bufs = (rows0_v, rows1_v, rows2_v)
    sems = (sem0, sem1, sem2)

    def gat(s, b):
        return pltpu.make_async_copy(tab_hbm.at[idx_v.at[s]], bufs[b], sems[b])

    def put(s, b):
        base = s * ROWS + wid * BCOL
        pltpu.sync_copy(bufs[b], out_hbm.at[pl.ds(base, BCOL)])

    for b in range(3):
        gat(b, b).start()

    def chunk_body(s3, carry):
        for b in range(3):
            s = s3 * 3 + b
            gat(s, b).wait()
            put(s, b)
            gat(s + 3, b).start()
        return carry

    lax.fori_loop(0, 65, chunk_body, 0)
    for s in range(195, 197):
        b = s % 3
        gat(s, b).wait()
        put(s, b)
        gat(s + 3, b).start()
    for s in range(197, 200):
        b = s % 3
        gat(s, b).wait()
        put(s, b)


def _finish_body(in_ref, out_ref):
    for s in range(8):
        for k in range(NW):
            r = s * NW + k
            x = in_ref[r * D:(r + 1) * D, :]
            t = x.T
            out_ref[s, :, k * BCOL:(k + 1) * BCOL] = jnp.concatenate(
                [t[0:D, :], t[D:2 * D, :]], axis=1) * jnp.float32(SCALE)


def _finish(gathered):
    return pl.pallas_call(
        _finish_body,
        grid=(COLS // 8,),
        in_specs=[pl.BlockSpec((ROWS * D * 8 // 128, 128), lambda s: (s, 0))],
        out_specs=pl.BlockSpec((8, D, ROWS), lambda s: (s, 0, 0)),
        out_shape=jax.ShapeDtypeStruct((COLS, D, ROWS), jnp.float32),
    )(gathered)


def kernel(x, table):
    xt = x.T.astype(jnp.int32)
    xs = ((xt & ~jnp.int32(127)) | ((xt & 63) << 1) | ((xt >> 6) & 1))
    xs = xs.reshape(COLS, NW, 2, D).transpose(0, 1, 3, 2).reshape(COLS, ROWS)

    tab_t = table.T
    packed = _pack_table(tab_t)
    tab_lin = packed.reshape(-1).reshape(2 * VPACK, D)

    gathered = _gather(xs, tab_lin)
    g2 = gathered.reshape(-1).reshape(ROWS * COLS * D // 128, 128)
    y = _finish(g2)
    return jnp.transpose(y, (2, 0, 1))
